# Initial kernel scaffold; baseline (speedup 1.0000x reference)
import jax
import jax.numpy as jnp
from jax.experimental import pallas as pl


def kernel(x, edge_index, edge_weight, weight, bias_param):
    raise NotImplementedError("write your pallas kernel here")


# speedup vs baseline: 3.6682x; 3.6682x over previous
import functools

import jax
import jax.numpy as jnp
from jax import lax
from jax.experimental import pallas as pl
from jax.experimental.pallas import tpu as pltpu
from jax.experimental.pallas import tpu_sc as plsc

NC = 2
NS = 16
NW = NC * NS
LANES = 16
CH = 128


def _make_hop(n_nodes, b, nchunk):
  mesh = plsc.VectorSubcoreMesh(core_axis_name="c", subcore_axis_name="s")
  rpt = n_nodes // NS

  @functools.partial(
      pl.kernel,
      out_type=jax.ShapeDtypeStruct((NC, n_nodes, b), jnp.float32),
      mesh=mesh,
      scratch_types=[
          pltpu.VMEM((nchunk, CH), jnp.int32),
          pltpu.VMEM((nchunk, CH), jnp.int32),
          pltpu.VMEM((nchunk, CH), jnp.float32),
          pltpu.VMEM((CH, b), jnp.float32),
          pltpu.VMEM_SHARED((n_nodes, b), jnp.float32),
      ],
      compiler_params=pltpu.CompilerParams(use_tc_tiling_on_sc=False,
                                           needs_layout_passes=False),
  )
  def hop(vt_hbm, src_hbm, dst_hbm, ew_hbm, z_hbm, out_hbm,
          src_v, dst_v, ew_v, rows_v, acc_sh):
    c = lax.axis_index("c")
    s = lax.axis_index("s")
    wid = s * NC + c

    pltpu.sync_copy(z_hbm, acc_sh.at[pl.ds(s * rpt, rpt)])

    pltpu.sync_copy(src_hbm.at[wid], src_v)
    pltpu.sync_copy(dst_hbm.at[wid], dst_v)
    pltpu.sync_copy(ew_hbm.at[wid], ew_v)
    plsc.subcore_barrier()

    lane_iota = lax.iota(jnp.int32, LANES)
    col_ids = [jnp.full((LANES,), f, jnp.int32) for f in range(b)]

    def chunk_body(j, carry):
      pltpu.sync_copy(vt_hbm.at[src_v.at[j]], rows_v)
      for g in range(CH // LANES):
        ridx = lane_iota + (g * LANES)
        ew16 = ew_v[j, pl.ds(g * LANES, LANES)]
        for f in range(b):
          vals = plsc.load_gather(rows_v, [ridx, col_ids[f]])
          plsc.store_scatter(rows_v, [ridx, col_ids[f]], vals * ew16)
      pltpu.sync_copy(rows_v, acc_sh.at[dst_v.at[j]], add=True)
      return carry

    lax.fori_loop(0, nchunk, chunk_body, 0)
    plsc.subcore_barrier()
    pltpu.sync_copy(acc_sh.at[pl.ds(s * rpt, rpt)],
                    out_hbm.at[c, pl.ds(s * rpt, rpt)])

  return hop


def _combine1(p, xt):
  def body(p_ref, x_ref, o_ref):
    o_ref[...] = p_ref[0] + p_ref[1] - x_ref[...]
  return pl.pallas_call(
      body,
      out_shape=jax.ShapeDtypeStruct(xt.shape, jnp.float32),
  )(p, xt)


def _final(xt, t1, q, weight, bias_param, nblk):
  n_nodes, b = xt.shape
  fsz = weight.shape[0]
  grid = n_nodes // nblk

  def body(x_ref, t1_ref, q_ref, w_ref, b_ref, o_ref):
    t0 = x_ref[...]
    t1v = t1_ref[...]
    t2 = 2.0 * (q_ref[0] + q_ref[1]) - 2.0 * t1v - t0
    w = w_ref[...]
    bias = b_ref[...]
    y = (t0.T[:, :, None] * w[None, None, :, 0]
         + t1v.T[:, :, None] * w[None, None, :, 1]
         + t2.T[:, :, None] * w[None, None, :, 2]
         + bias[None, None, :])
    o_ref[...] = y

  return pl.pallas_call(
      body,
      grid=(grid,),
      in_specs=[
          pl.BlockSpec((nblk, b), lambda i: (i, 0)),
          pl.BlockSpec((nblk, b), lambda i: (i, 0)),
          pl.BlockSpec((NC, nblk, b), lambda i: (0, i, 0)),
          pl.BlockSpec((fsz, 3), lambda i: (0, 0)),
          pl.BlockSpec((fsz,), lambda i: (0,)),
      ],
      out_specs=pl.BlockSpec((b, nblk, fsz), lambda i: (0, i, 0)),
      out_shape=jax.ShapeDtypeStruct((b, n_nodes, fsz), jnp.float32),
  )(xt, t1, q, weight, bias_param)


def kernel(x, edge_index, edge_weight, weight, bias_param):
  b, n_nodes = x.shape
  e = edge_weight.shape[0]

  epw = -(-e // (NW * CH)) * CH
  nchunk = epw // CH
  pad = epw * NW - e

  src = jnp.concatenate([edge_index[0], jnp.zeros((pad,), jnp.int32)])
  dst = jnp.concatenate([edge_index[1], jnp.zeros((pad,), jnp.int32)])
  ew = jnp.concatenate([edge_weight, jnp.zeros((pad,), jnp.float32)])
  src = src.reshape(NW, nchunk, CH)
  dst = dst.reshape(NW, nchunk, CH)
  ew = ew.reshape(NW, nchunk, CH)

  xt = x.T
  zblk = jnp.zeros((n_nodes // NS, b), jnp.float32)

  hop = _make_hop(n_nodes, b, nchunk)
  p = hop(xt, src, dst, ew, zblk)
  t1 = _combine1(p, xt)
  q = hop(t1, src, dst, ew, zblk)
  return _final(xt, t1, q, weight, bias_param, nblk=400)


# speedup vs baseline: 4.2414x; 1.1563x over previous
import functools

import jax
import jax.numpy as jnp
from jax import lax
from jax.experimental import pallas as pl
from jax.experimental.pallas import tpu as pltpu
from jax.experimental.pallas import tpu_sc as plsc

NC = 2
NS = 16
NW = NC * NS
LANES = 16
CH = 128


NBUF = 5
LEAD = 3


def _make_hop(n_nodes, b, nchunk):
  assert nchunk % NBUF == 0
  mesh = plsc.VectorSubcoreMesh(core_axis_name="c", subcore_axis_name="s")
  rpt = n_nodes // NS

  @functools.partial(
      pl.kernel,
      out_type=jax.ShapeDtypeStruct((NC, n_nodes, b), jnp.float32),
      mesh=mesh,
      scratch_types=[
          pltpu.VMEM((nchunk, CH), jnp.int32),
          pltpu.VMEM((nchunk, CH), jnp.int32),
          pltpu.VMEM((nchunk, CH), jnp.float32),
          [pltpu.VMEM((CH, b), jnp.float32) for _ in range(NBUF)],
          pltpu.VMEM_SHARED((n_nodes, b), jnp.float32),
          pltpu.SemaphoreType.DMA((NBUF,)),
          pltpu.SemaphoreType.DMA((NBUF,)),
      ],
      compiler_params=pltpu.CompilerParams(use_tc_tiling_on_sc=False,
                                           needs_layout_passes=False),
  )
  def hop(vt_hbm, src_hbm, dst_hbm, ew_hbm, z_hbm, out_hbm,
          src_v, dst_v, ew_v, bufs, acc_sh, gsem, ssem):
    c = lax.axis_index("c")
    s = lax.axis_index("s")
    wid = s * NC + c

    pltpu.sync_copy(z_hbm, acc_sh.at[pl.ds(s * rpt, rpt)])

    pltpu.sync_copy(src_hbm.at[wid], src_v)
    pltpu.sync_copy(dst_hbm.at[wid], dst_v)
    pltpu.sync_copy(ew_hbm.at[wid], ew_v)
    plsc.subcore_barrier()

    lane_iota = lax.iota(jnp.int32, LANES)
    col_ids = [jnp.full((LANES,), f, jnp.int32) for f in range(b)]

    def gather_start(j, bb):
      pltpu.async_copy(vt_hbm.at[src_v.at[j]], bufs[bb], gsem.at[bb])

    def gather_wait(j, bb):
      pltpu.make_async_copy(vt_hbm.at[src_v.at[j]], bufs[bb],
                            gsem.at[bb]).wait()

    def scatter_start(j, bb):
      pltpu.async_copy(bufs[bb], acc_sh.at[dst_v.at[j]], ssem.at[bb],
                       add=True)

    def scatter_wait(j, bb):
      pltpu.make_async_copy(bufs[bb], acc_sh.at[dst_v.at[j]],
                            ssem.at[bb]).wait()

    for j in range(LEAD):
      gather_start(j, j)

    def group_body(g, carry):
      for bb in range(NBUF):
        j = g * NBUF + bb
        gather_wait(j, bb)
        for gg in range(CH // LANES):
          ridx = lane_iota + (gg * LANES)
          ew16 = ew_v[j, pl.ds(gg * LANES, LANES)]
          for f in range(b):
            vals = plsc.load_gather(bufs[bb], [ridx, col_ids[f]])
            plsc.store_scatter(bufs[bb], [ridx, col_ids[f]], vals * ew16)
        scatter_start(j, bb)
        jn = j + LEAD
        bn = (bb + LEAD) % NBUF

        @pl.when(jn >= NBUF)
        def _():
          scatter_wait(jn - NBUF, bn)

        @pl.when(jn < nchunk)
        def _():
          gather_start(jn, bn)
      return carry

    lax.fori_loop(0, nchunk // NBUF, group_body, 0)
    for j in range(nchunk + LEAD - NBUF, nchunk):
      scatter_wait(j, j % NBUF)
    plsc.subcore_barrier()
    pltpu.sync_copy(acc_sh.at[pl.ds(s * rpt, rpt)],
                    out_hbm.at[c, pl.ds(s * rpt, rpt)])

  return hop


def _combine1(p, xt):
  def body(p_ref, x_ref, o_ref):
    o_ref[...] = p_ref[0] + p_ref[1] - x_ref[...]
  return pl.pallas_call(
      body,
      out_shape=jax.ShapeDtypeStruct(xt.shape, jnp.float32),
  )(p, xt)


def _final(xt, t1, q, weight, bias_param, nblk):
  n_nodes, b = xt.shape
  fsz = weight.shape[0]
  grid = n_nodes // nblk

  def body(x_ref, t1_ref, q_ref, w_ref, b_ref, o_ref):
    t0 = x_ref[...]
    t1v = t1_ref[...]
    t2 = 2.0 * (q_ref[0] + q_ref[1]) - 2.0 * t1v - t0
    w = w_ref[...]
    bias = b_ref[...]
    y = (t0.T[:, :, None] * w[None, None, :, 0]
         + t1v.T[:, :, None] * w[None, None, :, 1]
         + t2.T[:, :, None] * w[None, None, :, 2]
         + bias[None, None, :])
    o_ref[...] = y

  return pl.pallas_call(
      body,
      grid=(grid,),
      in_specs=[
          pl.BlockSpec((nblk, b), lambda i: (i, 0)),
          pl.BlockSpec((nblk, b), lambda i: (i, 0)),
          pl.BlockSpec((NC, nblk, b), lambda i: (0, i, 0)),
          pl.BlockSpec((fsz, 3), lambda i: (0, 0)),
          pl.BlockSpec((fsz,), lambda i: (0,)),
      ],
      out_specs=pl.BlockSpec((b, nblk, fsz), lambda i: (0, i, 0)),
      out_shape=jax.ShapeDtypeStruct((b, n_nodes, fsz), jnp.float32),
  )(xt, t1, q, weight, bias_param)


def kernel(x, edge_index, edge_weight, weight, bias_param):
  b, n_nodes = x.shape
  e = edge_weight.shape[0]

  epw = -(-e // (NW * CH * NBUF)) * CH * NBUF
  nchunk = epw // CH
  pad = epw * NW - e

  src = jnp.concatenate([edge_index[0], jnp.zeros((pad,), jnp.int32)])
  dst = jnp.concatenate([edge_index[1], jnp.zeros((pad,), jnp.int32)])
  ew = jnp.concatenate([edge_weight, jnp.zeros((pad,), jnp.float32)])
  src = src.reshape(NW, nchunk, CH)
  dst = dst.reshape(NW, nchunk, CH)
  ew = ew.reshape(NW, nchunk, CH)

  xt = x.T
  zblk = jnp.zeros((n_nodes // NS, b), jnp.float32)

  hop = _make_hop(n_nodes, b, nchunk)
  p = hop(xt, src, dst, ew, zblk)
  t1 = _combine1(p, xt)
  q = hop(t1, src, dst, ew, zblk)
  return _final(xt, t1, q, weight, bias_param, nblk=400)


# speedup vs baseline: 10.4387x; 2.4612x over previous
import functools

import jax
import jax.numpy as jnp
from jax import lax
from jax.experimental import pallas as pl
from jax.experimental.pallas import tpu as pltpu
from jax.experimental.pallas import tpu_sc as plsc

NC = 2
NS = 16
NW = NC * NS
LANES = 16
CH = 128


NBUF = 5
LEAD = 3


def _make_hop(n_nodes, b, nchunk):
  assert nchunk % NBUF == 0
  mesh = plsc.VectorSubcoreMesh(core_axis_name="c", subcore_axis_name="s")
  rpt = n_nodes // NS

  @functools.partial(
      pl.kernel,
      out_type=jax.ShapeDtypeStruct((NC, n_nodes, b), jnp.float32),
      mesh=mesh,
      scratch_types=[
          pltpu.VMEM((nchunk, CH), jnp.int32),
          pltpu.VMEM((nchunk, CH), jnp.int32),
          pltpu.VMEM((nchunk, CH), jnp.float32),
          [pltpu.VMEM((CH, b), jnp.float32) for _ in range(NBUF)],
          pltpu.VMEM_SHARED((n_nodes, b), jnp.float32),
          pltpu.SemaphoreType.DMA((NBUF,)),
          pltpu.SemaphoreType.DMA((NBUF,)),
      ],
      compiler_params=pltpu.CompilerParams(use_tc_tiling_on_sc=False,
                                           needs_layout_passes=False),
  )
  def hop(vt_hbm, src_hbm, dst_hbm, ew_hbm, z_hbm, out_hbm,
          src_v, dst_v, ew_v, bufs, acc_sh, gsem, ssem):
    c = lax.axis_index("c")
    s = lax.axis_index("s")
    wid = s * NC + c

    pltpu.sync_copy(z_hbm, acc_sh.at[pl.ds(s * rpt, rpt)])

    pltpu.sync_copy(src_hbm.at[wid], src_v)
    pltpu.sync_copy(dst_hbm.at[wid], dst_v)
    pltpu.sync_copy(ew_hbm.at[wid], ew_v)
    plsc.subcore_barrier()

    lane_iota = lax.iota(jnp.int32, LANES)
    col_ids = [jnp.full((LANES,), f, jnp.int32) for f in range(b)]

    def gather_start(j, bb):
      pltpu.async_copy(vt_hbm.at[src_v.at[j]], bufs[bb], gsem.at[bb])

    def gather_wait(j, bb):
      pltpu.make_async_copy(vt_hbm.at[src_v.at[j]], bufs[bb],
                            gsem.at[bb]).wait()

    def scatter_start(j, bb):
      pltpu.async_copy(bufs[bb], acc_sh.at[dst_v.at[j]], ssem.at[bb],
                       add=True)

    def scatter_wait(j, bb):
      pltpu.make_async_copy(bufs[bb], acc_sh.at[dst_v.at[j]],
                            ssem.at[bb]).wait()

    for j in range(LEAD):
      gather_start(j, j)

    def group_body(g, carry):
      for bb in range(NBUF):
        j = g * NBUF + bb
        gather_wait(j, bb)
        for gg in range(CH // LANES):
          ew16 = ew_v[j, pl.ds(gg * LANES, LANES)]
          for i in range(LANES):
            e = gg * LANES + i
            wv = jnp.full((LANES,), ew16[i])
            lo = bufs[bb][e, pl.ds(0, LANES)]
            hi = bufs[bb][e, pl.ds(LANES, LANES)]
            bufs[bb][e, pl.ds(0, LANES)] = lo * wv
            bufs[bb][e, pl.ds(LANES, LANES)] = hi * wv
        scatter_start(j, bb)
        jn = j + LEAD
        bn = (bb + LEAD) % NBUF

        @pl.when(jn >= NBUF)
        def _():
          scatter_wait(jn - NBUF, bn)

        @pl.when(jn < nchunk)
        def _():
          gather_start(jn, bn)
      return carry

    lax.fori_loop(0, nchunk // NBUF, group_body, 0)
    for j in range(nchunk + LEAD - NBUF, nchunk):
      scatter_wait(j, j % NBUF)
    plsc.subcore_barrier()
    pltpu.sync_copy(acc_sh.at[pl.ds(s * rpt, rpt)],
                    out_hbm.at[c, pl.ds(s * rpt, rpt)])

  return hop


def _combine1(p, xt):
  def body(p_ref, x_ref, o_ref):
    o_ref[...] = p_ref[0] + p_ref[1] - x_ref[...]
  return pl.pallas_call(
      body,
      out_shape=jax.ShapeDtypeStruct(xt.shape, jnp.float32),
  )(p, xt)


def _final(xt, t1, q, weight, bias_param, nblk):
  n_nodes, b = xt.shape
  fsz = weight.shape[0]
  grid = n_nodes // nblk

  def body(x_ref, t1_ref, q_ref, w_ref, b_ref, o_ref):
    t0 = x_ref[...]
    t1v = t1_ref[...]
    t2 = 2.0 * (q_ref[0] + q_ref[1]) - 2.0 * t1v - t0
    w = w_ref[...]
    bias = b_ref[...]
    y = (t0.T[:, :, None] * w[None, None, :, 0]
         + t1v.T[:, :, None] * w[None, None, :, 1]
         + t2.T[:, :, None] * w[None, None, :, 2]
         + bias[None, None, :])
    o_ref[...] = y

  return pl.pallas_call(
      body,
      grid=(grid,),
      in_specs=[
          pl.BlockSpec((nblk, b), lambda i: (i, 0)),
          pl.BlockSpec((nblk, b), lambda i: (i, 0)),
          pl.BlockSpec((NC, nblk, b), lambda i: (0, i, 0)),
          pl.BlockSpec((fsz, 3), lambda i: (0, 0)),
          pl.BlockSpec((fsz,), lambda i: (0,)),
      ],
      out_specs=pl.BlockSpec((b, nblk, fsz), lambda i: (0, i, 0)),
      out_shape=jax.ShapeDtypeStruct((b, n_nodes, fsz), jnp.float32),
  )(xt, t1, q, weight, bias_param)


def kernel(x, edge_index, edge_weight, weight, bias_param):
  b, n_nodes = x.shape
  e = edge_weight.shape[0]

  epw = -(-e // (NW * CH * NBUF)) * CH * NBUF
  nchunk = epw // CH
  pad = epw * NW - e

  src = jnp.concatenate([edge_index[0], jnp.zeros((pad,), jnp.int32)])
  dst = jnp.concatenate([edge_index[1], jnp.zeros((pad,), jnp.int32)])
  ew = jnp.concatenate([edge_weight, jnp.zeros((pad,), jnp.float32)])
  src = src.reshape(NW, nchunk, CH)
  dst = dst.reshape(NW, nchunk, CH)
  ew = ew.reshape(NW, nchunk, CH)

  xt = x.T
  zblk = jnp.zeros((n_nodes // NS, b), jnp.float32)

  hop = _make_hop(n_nodes, b, nchunk)
  p = hop(xt, src, dst, ew, zblk)
  t1 = _combine1(p, xt)
  q = hop(t1, src, dst, ew, zblk)
  return _final(xt, t1, q, weight, bias_param, nblk=400)


# speedup vs baseline: 13.6799x; 1.3105x over previous
import functools

import jax
import jax.numpy as jnp
from jax import lax
from jax.experimental import pallas as pl
from jax.experimental.pallas import tpu as pltpu
from jax.experimental.pallas import tpu_sc as plsc

NC = 2
NS = 16
NW = NC * NS
LANES = 16
CH = 128


NBUF = 5
LEAD = 3


def _make_hop(n_nodes, b, nchunk):
  assert nchunk % NBUF == 0
  mesh = plsc.VectorSubcoreMesh(core_axis_name="c", subcore_axis_name="s")
  rpt = n_nodes // NS

  @functools.partial(
      pl.kernel,
      out_type=jax.ShapeDtypeStruct((NC, n_nodes, b), jnp.float32),
      mesh=mesh,
      scratch_types=[
          pltpu.VMEM((nchunk, CH), jnp.int32),
          pltpu.VMEM((nchunk, CH), jnp.int32),
          pltpu.VMEM((nchunk, CH), jnp.float32),
          [pltpu.VMEM((CH, b), jnp.float32) for _ in range(NBUF)],
          pltpu.VMEM_SHARED((n_nodes, b), jnp.float32),
          pltpu.SemaphoreType.DMA((NBUF,)),
          pltpu.SemaphoreType.DMA((NBUF,)),
      ],
      compiler_params=pltpu.CompilerParams(use_tc_tiling_on_sc=False,
                                           needs_layout_passes=False),
  )
  def hop(vt_hbm, src_hbm, dst_hbm, ew_hbm, z_hbm, out_hbm,
          src_v, dst_v, ew_v, bufs, acc_sh, gsem, ssem):
    c = lax.axis_index("c")
    s = lax.axis_index("s")
    wid = s * NC + c

    pltpu.sync_copy(z_hbm, acc_sh.at[pl.ds(s * rpt, rpt)])

    pltpu.sync_copy(src_hbm.at[wid], src_v)
    pltpu.sync_copy(dst_hbm.at[wid], dst_v)
    pltpu.sync_copy(ew_hbm.at[wid], ew_v)
    plsc.subcore_barrier()

    lane_iota = lax.iota(jnp.int32, LANES)
    col_ids = [jnp.full((LANES,), f, jnp.int32) for f in range(b)]

    def gather_start(j, bb):
      pltpu.async_copy(vt_hbm.at[src_v.at[j]], bufs[bb], gsem.at[bb])

    def gather_wait(j, bb):
      pltpu.make_async_copy(vt_hbm.at[src_v.at[j]], bufs[bb],
                            gsem.at[bb]).wait()

    def scatter_start(j, bb):
      pltpu.async_copy(bufs[bb], acc_sh.at[dst_v.at[j]], ssem.at[bb],
                       add=True)

    def scatter_wait(j, bb):
      pltpu.make_async_copy(bufs[bb], acc_sh.at[dst_v.at[j]],
                            ssem.at[bb]).wait()

    for j in range(LEAD):
      gather_start(j, j)

    def group_body(g, carry):
      for bb in range(NBUF):
        j = g * NBUF + bb
        gather_wait(j, bb)
        for gg in range(CH // LANES):
          ew16 = ew_v[j, pl.ds(gg * LANES, LANES)]
          for i in range(LANES):
            e = gg * LANES + i
            wv = jnp.full((LANES,), ew16[i])
            lo = bufs[bb][e, pl.ds(0, LANES)]
            hi = bufs[bb][e, pl.ds(LANES, LANES)]
            bufs[bb][e, pl.ds(0, LANES)] = lo * wv
            bufs[bb][e, pl.ds(LANES, LANES)] = hi * wv
        scatter_start(j, bb)
        jn = j + LEAD
        bn = (bb + LEAD) % NBUF

        @pl.when(jn >= NBUF)
        def _():
          scatter_wait(jn - NBUF, bn)

        @pl.when(jn < nchunk)
        def _():
          gather_start(jn, bn)
      return carry

    lax.fori_loop(0, nchunk // NBUF, group_body, 0)
    for j in range(nchunk + LEAD - NBUF, nchunk):
      scatter_wait(j, j % NBUF)
    plsc.subcore_barrier()
    pltpu.sync_copy(acc_sh.at[pl.ds(s * rpt, rpt)],
                    out_hbm.at[c, pl.ds(s * rpt, rpt)])

  return hop


def _combine1(p, xt):
  def body(p_ref, x_ref, o_ref):
    o_ref[...] = p_ref[0] + p_ref[1] - x_ref[...]
  return pl.pallas_call(
      body,
      out_shape=jax.ShapeDtypeStruct(xt.shape, jnp.float32),
  )(p, xt)


def _combine2(q, t1, xt):
  def body(q_ref, t1_ref, x_ref, o_ref):
    o_ref[...] = (2.0 * (q_ref[0] + q_ref[1]) - 2.0 * t1_ref[...]
                  - x_ref[...])
  return pl.pallas_call(
      body,
      out_shape=jax.ShapeDtypeStruct(xt.shape, jnp.float32),
  )(q, t1, xt)


def _final(t0b, t1b, t2b, weight, bias_param, fblk):
  b, n_nodes = t0b.shape
  fsz = weight.shape[0]
  grid = fsz // fblk

  def body(t0_ref, t1_ref, t2_ref, w_ref, b_ref, o_ref):
    i = pl.program_id(0)
    w = w_ref[pl.ds(i * fblk, fblk), :]
    bias = b_ref[pl.ds(i * fblk, fblk), :]
    y = (t0_ref[...][:, None, :] * w[None, :, 0, None]
         + t1_ref[...][:, None, :] * w[None, :, 1, None]
         + t2_ref[...][:, None, :] * w[None, :, 2, None]
         + bias[None, :, :])
    o_ref[...] = y

  return pl.pallas_call(
      body,
      grid=(grid,),
      in_specs=[
          pl.BlockSpec((b, n_nodes), lambda i: (0, 0)),
          pl.BlockSpec((b, n_nodes), lambda i: (0, 0)),
          pl.BlockSpec((b, n_nodes), lambda i: (0, 0)),
          pl.BlockSpec((fsz, 3), lambda i: (0, 0)),
          pl.BlockSpec((fsz, 1), lambda i: (0, 0)),
      ],
      out_specs=pl.BlockSpec((b, fblk, n_nodes), lambda i: (0, i, 0)),
      out_shape=jax.ShapeDtypeStruct((b, fsz, n_nodes), jnp.float32),
  )(t0b, t1b, t2b, weight, bias_param.reshape(fsz, 1))


def kernel(x, edge_index, edge_weight, weight, bias_param):
  b, n_nodes = x.shape
  e = edge_weight.shape[0]

  epw = -(-e // (NW * CH * NBUF)) * CH * NBUF
  nchunk = epw // CH
  pad = epw * NW - e

  src = jnp.concatenate([edge_index[0], jnp.zeros((pad,), jnp.int32)])
  dst = jnp.concatenate([edge_index[1], jnp.zeros((pad,), jnp.int32)])
  ew = jnp.concatenate([edge_weight, jnp.zeros((pad,), jnp.float32)])
  src = src.reshape(NW, nchunk, CH)
  dst = dst.reshape(NW, nchunk, CH)
  ew = ew.reshape(NW, nchunk, CH)

  xt = x.T
  zblk = jnp.zeros((n_nodes // NS, b), jnp.float32)

  hop = _make_hop(n_nodes, b, nchunk)
  p = hop(xt, src, dst, ew, zblk)
  t1 = _combine1(p, xt)
  q = hop(t1, src, dst, ew, zblk)
  t2 = _combine2(q, t1, xt)
  yt = _final(x, t1.T, t2.T, weight, bias_param, fblk=8)
  return jnp.transpose(yt, (0, 2, 1))


# speedup vs baseline: 24.6705x; 1.8034x over previous
import functools

import jax
import jax.numpy as jnp
from jax import lax
from jax.experimental import pallas as pl
from jax.experimental.pallas import tpu as pltpu
from jax.experimental.pallas import tpu_sc as plsc

NC = 2
NS = 16
NW = NC * NS
LANES = 16
CH = 128


NBUF = 5
LEAD = 3


def _make_hop(n_nodes, b, nchunk):
  assert nchunk % NBUF == 0
  mesh = plsc.VectorSubcoreMesh(core_axis_name="c", subcore_axis_name="s")
  rpt = n_nodes // NS

  @functools.partial(
      pl.kernel,
      out_type=jax.ShapeDtypeStruct((NC, n_nodes, b), jnp.float32),
      mesh=mesh,
      scratch_types=[
          pltpu.VMEM((nchunk, CH), jnp.int32),
          pltpu.VMEM((nchunk, CH), jnp.int32),
          pltpu.VMEM((nchunk, CH), jnp.float32),
          [pltpu.VMEM((CH, b), jnp.float32) for _ in range(NBUF)],
          pltpu.VMEM_SHARED((n_nodes, b), jnp.float32),
          pltpu.VMEM_SHARED((n_nodes, b), jnp.float32),
          pltpu.SemaphoreType.DMA((NBUF,)),
          pltpu.SemaphoreType.DMA((NBUF,)),
      ],
      compiler_params=pltpu.CompilerParams(use_tc_tiling_on_sc=False,
                                           needs_layout_passes=False),
  )
  def hop(vt_hbm, src_hbm, dst_hbm, ew_hbm, z_hbm, out_hbm,
          src_v, dst_v, ew_v, bufs, acc_sh, tab_sh, gsem, ssem):
    c = lax.axis_index("c")
    s = lax.axis_index("s")
    wid = s * NC + c

    pltpu.sync_copy(z_hbm, acc_sh.at[pl.ds(s * rpt, rpt)])
    pltpu.sync_copy(vt_hbm.at[pl.ds(s * rpt, rpt)],
                    tab_sh.at[pl.ds(s * rpt, rpt)])

    pltpu.sync_copy(src_hbm.at[wid], src_v)
    pltpu.sync_copy(dst_hbm.at[wid], dst_v)
    pltpu.sync_copy(ew_hbm.at[wid], ew_v)
    plsc.subcore_barrier()

    lane_iota = lax.iota(jnp.int32, LANES)
    col_ids = [jnp.full((LANES,), f, jnp.int32) for f in range(b)]

    def gather_start(j, bb):
      pltpu.async_copy(tab_sh.at[src_v.at[j]], bufs[bb], gsem.at[bb])

    def gather_wait(j, bb):
      pltpu.make_async_copy(tab_sh.at[src_v.at[j]], bufs[bb],
                            gsem.at[bb]).wait()

    def scatter_start(j, bb):
      pltpu.async_copy(bufs[bb], acc_sh.at[dst_v.at[j]], ssem.at[bb],
                       add=True)

    def scatter_wait(j, bb):
      pltpu.make_async_copy(bufs[bb], acc_sh.at[dst_v.at[j]],
                            ssem.at[bb]).wait()

    for j in range(LEAD):
      gather_start(j, j)

    def group_body(g, carry):
      for bb in range(NBUF):
        j = g * NBUF + bb
        gather_wait(j, bb)
        for gg in range(CH // LANES):
          ew16 = ew_v[j, pl.ds(gg * LANES, LANES)]
          for i in range(LANES):
            e = gg * LANES + i
            wv = jnp.full((LANES,), ew16[i])
            lo = bufs[bb][e, pl.ds(0, LANES)]
            hi = bufs[bb][e, pl.ds(LANES, LANES)]
            bufs[bb][e, pl.ds(0, LANES)] = lo * wv
            bufs[bb][e, pl.ds(LANES, LANES)] = hi * wv
        scatter_start(j, bb)
        jn = j + LEAD
        bn = (bb + LEAD) % NBUF

        @pl.when(jn >= NBUF)
        def _():
          scatter_wait(jn - NBUF, bn)

        @pl.when(jn < nchunk)
        def _():
          gather_start(jn, bn)
      return carry

    lax.fori_loop(0, nchunk // NBUF, group_body, 0)
    for j in range(nchunk + LEAD - NBUF, nchunk):
      scatter_wait(j, j % NBUF)
    plsc.subcore_barrier()
    pltpu.sync_copy(acc_sh.at[pl.ds(s * rpt, rpt)],
                    out_hbm.at[c, pl.ds(s * rpt, rpt)])

  return hop


def _combine1(p, xt):
  def body(p_ref, x_ref, o_ref):
    o_ref[...] = p_ref[0] + p_ref[1] - x_ref[...]
  return pl.pallas_call(
      body,
      out_shape=jax.ShapeDtypeStruct(xt.shape, jnp.float32),
  )(p, xt)


def _combine2(q, t1, xt):
  def body(q_ref, t1_ref, x_ref, o_ref):
    o_ref[...] = (2.0 * (q_ref[0] + q_ref[1]) - 2.0 * t1_ref[...]
                  - x_ref[...])
  return pl.pallas_call(
      body,
      out_shape=jax.ShapeDtypeStruct(xt.shape, jnp.float32),
  )(q, t1, xt)


def _final(t0b, t1b, t2b, weight, bias_param, fblk):
  b, n_nodes = t0b.shape
  fsz = weight.shape[0]
  grid = fsz // fblk

  def body(t0_ref, t1_ref, t2_ref, w_ref, b_ref, o_ref):
    i = pl.program_id(0)
    w = w_ref[pl.ds(i * fblk, fblk), :]
    bias = b_ref[pl.ds(i * fblk, fblk), :]
    y = (t0_ref[...][:, None, :] * w[None, :, 0, None]
         + t1_ref[...][:, None, :] * w[None, :, 1, None]
         + t2_ref[...][:, None, :] * w[None, :, 2, None]
         + bias[None, :, :])
    o_ref[...] = y

  return pl.pallas_call(
      body,
      grid=(grid,),
      in_specs=[
          pl.BlockSpec((b, n_nodes), lambda i: (0, 0)),
          pl.BlockSpec((b, n_nodes), lambda i: (0, 0)),
          pl.BlockSpec((b, n_nodes), lambda i: (0, 0)),
          pl.BlockSpec((fsz, 3), lambda i: (0, 0)),
          pl.BlockSpec((fsz, 1), lambda i: (0, 0)),
      ],
      out_specs=pl.BlockSpec((b, fblk, n_nodes), lambda i: (0, i, 0)),
      out_shape=jax.ShapeDtypeStruct((b, fsz, n_nodes), jnp.float32),
  )(t0b, t1b, t2b, weight, bias_param.reshape(fsz, 1))


def kernel(x, edge_index, edge_weight, weight, bias_param):
  b, n_nodes = x.shape
  e = edge_weight.shape[0]

  epw = -(-e // (NW * CH * NBUF)) * CH * NBUF
  nchunk = epw // CH
  pad = epw * NW - e

  src = jnp.concatenate([edge_index[0], jnp.zeros((pad,), jnp.int32)])
  dst = jnp.concatenate([edge_index[1], jnp.zeros((pad,), jnp.int32)])
  ew = jnp.concatenate([edge_weight, jnp.zeros((pad,), jnp.float32)])
  src = src.reshape(NW, nchunk, CH)
  dst = dst.reshape(NW, nchunk, CH)
  ew = ew.reshape(NW, nchunk, CH)

  xt = x.T
  zblk = jnp.zeros((n_nodes // NS, b), jnp.float32)

  hop = _make_hop(n_nodes, b, nchunk)
  p = hop(xt, src, dst, ew, zblk)
  t1 = _combine1(p, xt)
  q = hop(t1, src, dst, ew, zblk)
  t2 = _combine2(q, t1, xt)
  yt = _final(x, t1.T, t2.T, weight, bias_param, fblk=8)
  return jnp.transpose(yt, (0, 2, 1))


# speedup vs baseline: 25.9706x; 1.0527x over previous
import functools

import jax
import jax.numpy as jnp
from jax import lax
from jax.experimental import pallas as pl
from jax.experimental.pallas import tpu as pltpu
from jax.experimental.pallas import tpu_sc as plsc

NC = 2
NS = 16
NW = NC * NS
LANES = 16
CH = 128


NBUF = 5
LEAD = 3


def _make_hop(n_nodes, b, nchunk):
  assert nchunk % NBUF == 0
  mesh = plsc.VectorSubcoreMesh(core_axis_name="c", subcore_axis_name="s")
  rpt = n_nodes // NS

  @functools.partial(
      pl.kernel,
      out_type=jax.ShapeDtypeStruct((NC, n_nodes, b), jnp.float32),
      mesh=mesh,
      scratch_types=[
          pltpu.VMEM((nchunk, CH), jnp.int32),
          pltpu.VMEM((nchunk, CH), jnp.int32),
          pltpu.VMEM((nchunk, CH), jnp.float32),
          [pltpu.VMEM((CH, b), jnp.float32) for _ in range(NBUF)],
          pltpu.VMEM_SHARED((n_nodes, b), jnp.float32),
          pltpu.VMEM_SHARED((n_nodes, b), jnp.float32),
          pltpu.SemaphoreType.DMA((NBUF,)),
          pltpu.SemaphoreType.DMA((NBUF,)),
      ],
      compiler_params=pltpu.CompilerParams(use_tc_tiling_on_sc=False,
                                           needs_layout_passes=False),
  )
  def hop(vt_hbm, src_hbm, dst_hbm, ew_hbm, z_hbm, out_hbm,
          src_v, dst_v, ew_v, bufs, acc_sh, tab_sh, gsem, ssem):
    c = lax.axis_index("c")
    s = lax.axis_index("s")
    wid = s * NC + c

    pltpu.sync_copy(z_hbm, acc_sh.at[pl.ds(s * rpt, rpt)])
    pltpu.sync_copy(vt_hbm.at[pl.ds(s * rpt, rpt)],
                    tab_sh.at[pl.ds(s * rpt, rpt)])

    pltpu.sync_copy(src_hbm.at[wid], src_v)
    pltpu.sync_copy(dst_hbm.at[wid], dst_v)
    pltpu.sync_copy(ew_hbm.at[wid], ew_v)
    plsc.subcore_barrier()

    lane_iota = lax.iota(jnp.int32, LANES)
    col_ids = [jnp.full((LANES,), f, jnp.int32) for f in range(b)]

    def gather_start(j, bb):
      pltpu.async_copy(tab_sh.at[src_v.at[j]], bufs[bb], gsem.at[bb])

    def gather_wait(j, bb):
      pltpu.make_async_copy(tab_sh.at[src_v.at[j]], bufs[bb],
                            gsem.at[bb]).wait()

    def scatter_start(j, bb):
      pltpu.async_copy(bufs[bb], acc_sh.at[dst_v.at[j]], ssem.at[bb],
                       add=True)

    def scatter_wait(j, bb):
      pltpu.make_async_copy(bufs[bb], acc_sh.at[dst_v.at[j]],
                            ssem.at[bb]).wait()

    for j in range(LEAD):
      gather_start(j, j)

    def group_body(g, carry):
      for bb in range(NBUF):
        j = g * NBUF + bb
        gather_wait(j, bb)
        for gg in range(CH // LANES):
          ew16 = ew_v[j, pl.ds(gg * LANES, LANES)]
          for i in range(LANES):
            e = gg * LANES + i
            wv = jnp.full((LANES,), ew16[i])
            lo = bufs[bb][e, pl.ds(0, LANES)]
            hi = bufs[bb][e, pl.ds(LANES, LANES)]
            bufs[bb][e, pl.ds(0, LANES)] = lo * wv
            bufs[bb][e, pl.ds(LANES, LANES)] = hi * wv
        scatter_start(j, bb)
        jn = j + LEAD
        bn = (bb + LEAD) % NBUF

        @pl.when(jn >= NBUF)
        def _():
          scatter_wait(jn - NBUF, bn)

        @pl.when(jn < nchunk)
        def _():
          gather_start(jn, bn)
      return carry

    lax.fori_loop(0, nchunk // NBUF, group_body, 0)
    for j in range(nchunk + LEAD - NBUF, nchunk):
      scatter_wait(j, j % NBUF)
    plsc.subcore_barrier()
    pltpu.sync_copy(acc_sh.at[pl.ds(s * rpt, rpt)],
                    out_hbm.at[c, pl.ds(s * rpt, rpt)])

  return hop


def _combine1(p, x, xt):
  def body(p_ref, x_ref, xt_ref, t1_ref, t1b_ref):
    psum = p_ref[0] + p_ref[1]
    t1_ref[...] = psum - xt_ref[...]
    t1b_ref[...] = psum.T - x_ref[...]
  return pl.pallas_call(
      body,
      out_shape=(jax.ShapeDtypeStruct(xt.shape, jnp.float32),
                 jax.ShapeDtypeStruct(x.shape, jnp.float32)),
  )(p, x, xt)


def _combine2(q, t1b, x):
  def body(q_ref, t1b_ref, x_ref, o_ref):
    o_ref[...] = (2.0 * (q_ref[0] + q_ref[1]).T - 2.0 * t1b_ref[...]
                  - x_ref[...])
  return pl.pallas_call(
      body,
      out_shape=jax.ShapeDtypeStruct(x.shape, jnp.float32),
  )(q, t1b, x)


def _final(t0b, t1b, t2b, weight, bias_param, fblk):
  b, n_nodes = t0b.shape
  fsz = weight.shape[0]
  grid = fsz // fblk

  def body(t0_ref, t1_ref, t2_ref, w_ref, b_ref, o_ref):
    i = pl.program_id(0)
    t0 = t0_ref[...]
    t1 = t1_ref[...]
    t2 = t2_ref[...]
    for f in range(fblk):
      o_ref[:, f, :] = (t0 * w_ref[i * fblk + f, 0]
                        + t1 * w_ref[i * fblk + f, 1]
                        + t2 * w_ref[i * fblk + f, 2]
                        + b_ref[i * fblk + f, 0])

  return pl.pallas_call(
      body,
      grid=(grid,),
      in_specs=[
          pl.BlockSpec((b, n_nodes), lambda i: (0, 0)),
          pl.BlockSpec((b, n_nodes), lambda i: (0, 0)),
          pl.BlockSpec((b, n_nodes), lambda i: (0, 0)),
          pl.BlockSpec((fsz, 3), lambda i: (0, 0)),
          pl.BlockSpec((fsz, 1), lambda i: (0, 0)),
      ],
      out_specs=pl.BlockSpec((b, fblk, n_nodes), lambda i: (0, i, 0)),
      out_shape=jax.ShapeDtypeStruct((b, fsz, n_nodes), jnp.float32),
  )(t0b, t1b, t2b, weight, bias_param.reshape(fsz, 1))


def kernel(x, edge_index, edge_weight, weight, bias_param):
  b, n_nodes = x.shape
  e = edge_weight.shape[0]

  epw = -(-e // (NW * CH * NBUF)) * CH * NBUF
  nchunk = epw // CH
  pad = epw * NW - e

  src = jnp.concatenate([edge_index[0], jnp.zeros((pad,), jnp.int32)])
  dst = jnp.concatenate([edge_index[1], jnp.zeros((pad,), jnp.int32)])
  ew = jnp.concatenate([edge_weight, jnp.zeros((pad,), jnp.float32)])
  src = src.reshape(NW, nchunk, CH)
  dst = dst.reshape(NW, nchunk, CH)
  ew = ew.reshape(NW, nchunk, CH)

  xt = x.T
  zblk = jnp.zeros((n_nodes // NS, b), jnp.float32)

  hop = _make_hop(n_nodes, b, nchunk)
  p = hop(xt, src, dst, ew, zblk)
  t1, t1b = _combine1(p, x, xt)
  q = hop(t1, src, dst, ew, zblk)
  t2b = _combine2(q, t1b, x)
  yt = _final(x, t1b, t2b, weight, bias_param, fblk=8)
  return jnp.transpose(yt, (0, 2, 1))
